# baseline (speedup 1.0000x reference)
import functools

import jax
import jax.numpy as jnp
from jax import lax
from jax.experimental import pallas as pl
from jax.experimental.pallas import tpu as pltpu
from jax.experimental.pallas import tpu_sc as plsc

B, P, A, D = 2048, 32, 16, 64
BP = B * P
N_U = N_R = N_A = 100000

_NC, _NS = 2, 16
_NW = _NC * _NS
_PPW = BP // _NW
_C = 16
_NCHUNK = _PPW // _C
_NODES_PW = B // _NW


_CP = 64
_NCHUNK_P = _PPW // _CP


def _sc_paths(paths_nat, nodes, u2e, r2e):
  mesh = plsc.VectorSubcoreMesh(core_axis_name="c", subcore_axis_name="s")

  @functools.partial(
      pl.kernel,
      out_type=(
          jax.ShapeDtypeStruct((BP, D), jnp.float32),
          jax.ShapeDtypeStruct((BP, D), jnp.float32),
          jax.ShapeDtypeStruct((BP, D), jnp.float32),
          jax.ShapeDtypeStruct((B, D), jnp.float32),
      ),
      mesh=mesh,
      compiler_params=pltpu.CompilerParams(use_tc_tiling_on_sc=False,
                                           needs_layout_passes=False),
      scratch_types=[
          pltpu.VMEM((3 * P, _NODES_PW), jnp.int32),
          pltpu.VMEM((_PPW,), jnp.int32),
          pltpu.VMEM((_PPW,), jnp.int32),
          pltpu.VMEM((_PPW,), jnp.int32),
          pltpu.VMEM((3, _CP, D), jnp.float32),
          pltpu.VMEM((3, _CP, D), jnp.float32),
          pltpu.VMEM((3, _CP, D), jnp.float32),
          pltpu.VMEM((_NODES_PW,), jnp.int32),
          pltpu.VMEM((_NODES_PW, D), jnp.float32),
      ] + [pltpu.SemaphoreType.DMA] * 19,
  )
  def k(paths_h, nodes_h, u2e_h, r2e_h, r1_o, r2_o, ng_o, self_o,
        pblk, r1a, r2a, nga, b1v, b2v, b3v, sidx, srows, *sems):
    gsem = [sems[0:3], sems[3:6], sems[6:9]]
    osem = [sems[9:12], sems[12:15], sems[15:18]]
    ssem = sems[18]
    wid = lax.axis_index("s") * _NC + lax.axis_index("c")
    pbase = wid * _PPW
    nbase = wid * _NODES_PW

    pltpu.sync_copy(nodes_h.at[pl.ds(nbase, _NODES_PW)], sidx)
    scp = pltpu.async_copy(u2e_h.at[sidx], srows, ssem)
    pltpu.sync_copy(paths_h.at[:, pl.ds(nbase, _NODES_PW)], pblk)

    def deint(h, _):
      qq = lax.iota(jnp.int32, 16) + h * 16
      prow = qq & (P - 1)
      bcol = qq >> 5
      r1a[pl.ds(h * 16, 16)] = plsc.load_gather(pblk, [prow, bcol])
      r2a[pl.ds(h * 16, 16)] = plsc.load_gather(pblk, [prow + P, bcol])
      nga[pl.ds(h * 16, 16)] = plsc.load_gather(pblk, [prow + 2 * P, bcol])
      return 0

    lax.fori_loop(0, _PPW // 16, deint, 0)
    scp.wait()
    pltpu.sync_copy(srows, self_o.at[pl.ds(nbase, _NODES_PW)])

    def issue(c, s):
      g = c * _CP
      pltpu.async_copy(r2e_h.at[r1a.at[pl.ds(g, _CP)]], b1v.at[s],
                       gsem[s][0])
      pltpu.async_copy(r2e_h.at[r2a.at[pl.ds(g, _CP)]], b2v.at[s],
                       gsem[s][1])
      pltpu.async_copy(u2e_h.at[nga.at[pl.ds(g, _CP)]], b3v.at[s],
                       gsem[s][2])

    def wait_gathers(s):
      pltpu.make_async_copy(r2e_h.at[r1a.at[pl.ds(0, _CP)]], b1v.at[s],
                            gsem[s][0]).wait()
      pltpu.make_async_copy(r2e_h.at[r2a.at[pl.ds(0, _CP)]], b2v.at[s],
                            gsem[s][1]).wait()
      pltpu.make_async_copy(u2e_h.at[nga.at[pl.ds(0, _CP)]], b3v.at[s],
                            gsem[s][2]).wait()

    def writeout(c, s):
      g = pbase + c * _CP
      pltpu.async_copy(b1v.at[s], r1_o.at[pl.ds(g, _CP)], osem[s][0])
      pltpu.async_copy(b2v.at[s], r2_o.at[pl.ds(g, _CP)], osem[s][1])
      pltpu.async_copy(b3v.at[s], ng_o.at[pl.ds(g, _CP)], osem[s][2])

    def wait_out(s):
      pltpu.make_async_copy(b1v.at[s], r1_o.at[pl.ds(0, _CP)],
                            osem[s][0]).wait()
      pltpu.make_async_copy(b2v.at[s], r2_o.at[pl.ds(0, _CP)],
                            osem[s][1]).wait()
      pltpu.make_async_copy(b3v.at[s], ng_o.at[pl.ds(0, _CP)],
                            osem[s][2]).wait()

    def chunk_step(c, s):
      wait_gathers(s)
      writeout(c, s)
      s2 = (s + 2) % 3

      def launch_next():
        pl.when(c + 2 >= 3)(lambda: wait_out(s2))
        issue(c + 2, s2)

      pl.when(c + 2 < _NCHUNK_P)(launch_next)

    issue(0, 0)
    issue(1, 1)

    def body(i, _):
      for s in range(3):
        chunk_step(3 * i + s, s)
      return 0

    lax.fori_loop(0, _NCHUNK_P // 3, body, 0)
    for c in range(_NCHUNK_P - _NCHUNK_P % 3, _NCHUNK_P):
      chunk_step(jnp.int32(c), c % 3)
    for s in range(3):
      wait_out(s)

  return k(paths_nat, nodes, u2e, r2e)


def _sc_attrs(attrs_nat, ua2e):
  mesh = plsc.VectorSubcoreMesh(core_axis_name="c", subcore_axis_name="s")

  @functools.partial(
      pl.kernel,
      out_type=jax.ShapeDtypeStruct((BP, D), jnp.float32),
      mesh=mesh,
      compiler_params=pltpu.CompilerParams(use_tc_tiling_on_sc=False,
                                           needs_layout_passes=False),
      scratch_types=[
          pltpu.VMEM((P * A, _NODES_PW), jnp.int32),
          pltpu.VMEM((5, _C * A), jnp.int32),
          pltpu.VMEM((5, _C * A, D), jnp.float32),
          pltpu.VMEM((5, _C, D), jnp.float32),
      ] + [pltpu.SemaphoreType.DMA] * 10,
  )
  def k(attr_h, ua2e_h, at_o, ablk, iav, bav, accv, *sems):
    gsem = sems[0:5]
    osem = sems[5:10]
    wid = lax.axis_index("s") * _NC + lax.axis_index("c")
    pbase = wid * _PPW
    nbase = wid * _NODES_PW

    pltpu.sync_copy(attr_h.at[:, pl.ds(nbase, _NODES_PW)], ablk)

    def issue(c, s):
      g = c * _C
      def build_attr_idx(h, _):
        qq = lax.iota(jnp.int32, 16) + g * A + h * 16
        iav[s, pl.ds(h * 16, 16)] = plsc.load_gather(
            ablk, [qq & (P * A - 1), qq >> 9])
        return 0

      lax.fori_loop(0, _C * A // 16, build_attr_idx, 0)
      pltpu.async_copy(ua2e_h.at[iav.at[s]], bav.at[s], gsem[s])

    def wait_gathers(s):
      pltpu.make_async_copy(ua2e_h.at[iav.at[s]], bav.at[s], gsem[s]).wait()

    def process(s):
      def path_body(p, _):
        base = p * A
        for c4 in range(D // 16):
          col = pl.ds(c4 * 16, 16)
          acc = bav[s, base, col]
          for r in range(1, A):
            acc = acc + bav[s, base + r, col]
          accv[s, p, col] = acc
        return 0

      lax.fori_loop(0, _C, path_body, 0)

    def writeout(c, s):
      g = pbase + c * _C
      pltpu.async_copy(accv.at[s], at_o.at[pl.ds(g, _C)], osem[s])

    def wait_out(s):
      pltpu.make_async_copy(accv.at[s], at_o.at[pl.ds(0, _C)],
                            osem[s]).wait()

    def chunk_step(c, s):
      wait_gathers(s)
      process(s)
      writeout(c, s)
      s2 = (s + 4) % 5

      def launch_next():
        pl.when(c + 4 >= 5)(lambda: wait_out(s2))
        issue(c + 4, s2)

      pl.when(c + 4 < _NCHUNK)(launch_next)

    for i in range(4):
      issue(i, i)

    def body(i, _):
      for s in range(5):
        chunk_step(5 * i + s, s)
      return 0

    lax.fori_loop(0, _NCHUNK // 5, body, 0)
    for c in range(_NCHUNK - _NCHUNK % 5, _NCHUNK):
      chunk_step(jnp.int32(c), c % 5)
    for s in range(5):
      wait_out(s)

  return k(attrs_nat, ua2e)


_NB = 128
_R = _NB * P


_R2 = _NB * P // 2


def _tc_body(r1_ref, r2_ref, ng_ref, at_ref, self_ref, w1_ref, b1_ref,
             w2_ref, b2_ref, wa1_ref, ba1_ref, wa2_ref, ba2_ref, wa3_ref,
             out_ref):
  f32 = jnp.float32

  def dot(a, b):
    return jnp.dot(a.astype(jnp.bfloat16), b.astype(jnp.bfloat16),
                   preferred_element_type=f32)

  rr = 2 * _R2

  def unpair(ref):
    x = ref[...]
    return jnp.concatenate([x[:, 0:D], x[:, D:2 * D]], axis=0)

  x1, x2, x3, x4 = (unpair(r1_ref), unpair(r2_ref), unpair(ng_ref),
                    unpair(at_ref))
  w1 = w1_ref[...]
  h1 = (dot(x1, w1[0:D, :]) + dot(x2, w1[D:2 * D, :]) +
        dot(x3, w1[2 * D:3 * D, :]) + dot(x4, w1[3 * D:4 * D, :]) +
        b1_ref[...])
  h1 = jnp.maximum(h1, 0.0)
  o = jnp.maximum(dot(h1, w2_ref[...]) + b2_ref[...], 0.0)

  node_of = lambda r: (r % _R2) // (P // 2)
  ind = (node_of(lax.broadcasted_iota(jnp.int32, (_NB, rr), 1)) ==
         lax.broadcasted_iota(jnp.int32, (_NB, rr), 0)).astype(f32)
  indT = (node_of(lax.broadcasted_iota(jnp.int32, (rr, _NB), 0)) ==
          lax.broadcasted_iota(jnp.int32, (rr, _NB), 1)).astype(f32)

  wa1 = wa1_ref[...]
  self_w = dot(self_ref[...], wa1[D:2 * D, :])
  a1 = jnp.maximum(dot(o, wa1[0:D, :]) + dot(indT, self_w) + ba1_ref[...],
                   0.0)
  a2 = jnp.maximum(dot(a1, wa2_ref[...]) + ba2_ref[...], 0.0)
  logit = dot(a2, wa3_ref[...])
  e = jnp.exp(logit - jnp.max(logit))
  num = dot(ind, o * e)
  den = dot(ind, e)
  out_ref[...] = num / den


def _tc_dense(r1_es, r2_es, ng_es, at_es, self_e, W1, b1, W2, b2, Wa1, ba1,
              Wa2, ba2, Wa3):
  grid = (B // _NB,)
  pair_spec = pl.BlockSpec((_R2, 2 * D), lambda i: (i, 0))
  node_spec = pl.BlockSpec((_NB, D), lambda i: (i, 0))

  def full(shape):
    return pl.BlockSpec(shape, lambda i: tuple(0 for _ in shape))

  return pl.pallas_call(
      _tc_body,
      grid=grid,
      in_specs=[
          pair_spec, pair_spec, pair_spec, pair_spec, node_spec,
          full((4 * D, 2 * D)), full((1, 2 * D)),
          full((2 * D, D)), full((1, D)),
          full((2 * D, D)), full((1, D)),
          full((D, D)), full((1, D)),
          full((D, 1)),
      ],
      out_specs=node_spec,
      out_shape=jax.ShapeDtypeStruct((B, D), jnp.float32),
  )(r1_es.reshape(BP // 2, 2 * D), r2_es.reshape(BP // 2, 2 * D),
    ng_es.reshape(BP // 2, 2 * D), at_es.reshape(BP // 2, 2 * D),
    self_e, W1, b1.reshape(1, -1), W2, b2.reshape(1, -1), Wa1,
    ba1.reshape(1, -1), Wa2, ba2.reshape(1, -1), Wa3)


@jax.jit
def kernel(nodes, nodes_l2paths, nodes_l2n_attrs, u2e, r2e, ua2e, W1, b1,
           W2, b2, Wa1, ba1, Wa2, ba2, Wa3, ba3):
  paths_nat = nodes_l2paths.transpose(2, 1, 0).reshape(3 * P, B).astype(
      jnp.int32)
  attrs_nat = nodes_l2n_attrs.transpose(1, 2, 0).reshape(P * A, B).astype(
      jnp.int32)
  nodes32 = nodes.reshape(-1).astype(jnp.int32)
  u2e_l = u2e.reshape(-1).reshape(N_U, D)
  r2e_l = r2e.reshape(-1).reshape(N_R, D)
  ua2e_l = ua2e.reshape(-1).reshape(N_A, D)

  r1_es, r2_es, ng_es, self_e = _sc_paths(paths_nat, nodes32, u2e_l, r2e_l)
  attrs_nat = lax.optimization_barrier((attrs_nat, r1_es[0, 0]))[0]
  at_es = _sc_attrs(attrs_nat, ua2e_l)
  del ba3
  return _tc_dense(r1_es, r2_es, ng_es, at_es, self_e, W1, b1, W2, b2,
                   Wa1, ba1, Wa2, ba2, Wa3)


# speedup vs baseline: 1.1097x; 1.1097x over previous
import functools

import jax
import jax.numpy as jnp
from jax import lax
from jax.experimental import pallas as pl
from jax.experimental.pallas import tpu as pltpu
from jax.experimental.pallas import tpu_sc as plsc

B, P, A, D = 2048, 32, 16, 64
BP = B * P
N_U = N_R = N_A = 100000

_NC, _NS = 2, 16
_NW = _NC * _NS
_PPW = BP // _NW
_C = 16
_NCHUNK = _PPW // _C
_NODES_PW = B // _NW


_CP = 64
_NCHUNK_P = _PPW // _CP


def _sc_paths(paths_nat, nodes, u2e, r2e):
  mesh = plsc.VectorSubcoreMesh(core_axis_name="c", subcore_axis_name="s")

  @functools.partial(
      pl.kernel,
      out_type=(
          jax.ShapeDtypeStruct((BP, D), jnp.float32),
          jax.ShapeDtypeStruct((BP, D), jnp.float32),
          jax.ShapeDtypeStruct((BP, D), jnp.float32),
          jax.ShapeDtypeStruct((B, D), jnp.float32),
      ),
      mesh=mesh,
      compiler_params=pltpu.CompilerParams(use_tc_tiling_on_sc=False,
                                           needs_layout_passes=False),
      scratch_types=[
          pltpu.VMEM((3 * P, _NODES_PW), jnp.int32),
          pltpu.VMEM((_PPW,), jnp.int32),
          pltpu.VMEM((_PPW,), jnp.int32),
          pltpu.VMEM((_PPW,), jnp.int32),
          pltpu.VMEM((3, _CP, D), jnp.float32),
          pltpu.VMEM((3, _CP, D), jnp.float32),
          pltpu.VMEM((3, _CP, D), jnp.float32),
          pltpu.VMEM((_NODES_PW,), jnp.int32),
          pltpu.VMEM((_NODES_PW, D), jnp.float32),
      ] + [pltpu.SemaphoreType.DMA] * 19,
  )
  def k(paths_h, nodes_h, u2e_h, r2e_h, r1_o, r2_o, ng_o, self_o,
        pblk, r1a, r2a, nga, b1v, b2v, b3v, sidx, srows, *sems):
    gsem = [sems[0:3], sems[3:6], sems[6:9]]
    osem = [sems[9:12], sems[12:15], sems[15:18]]
    ssem = sems[18]
    wid = lax.axis_index("s") * _NC + lax.axis_index("c")
    pbase = wid * _PPW
    nbase = wid * _NODES_PW

    pltpu.sync_copy(nodes_h.at[pl.ds(nbase, _NODES_PW)], sidx)
    scp = pltpu.async_copy(u2e_h.at[sidx], srows, ssem)
    pltpu.sync_copy(paths_h.at[:, pl.ds(nbase, _NODES_PW)], pblk)

    def deint(h, _):
      qq = lax.iota(jnp.int32, 16) + h * 16
      prow = qq & (P - 1)
      bcol = qq >> 5
      r1a[pl.ds(h * 16, 16)] = plsc.load_gather(pblk, [prow, bcol])
      r2a[pl.ds(h * 16, 16)] = plsc.load_gather(pblk, [prow + P, bcol])
      nga[pl.ds(h * 16, 16)] = plsc.load_gather(pblk, [prow + 2 * P, bcol])
      return 0

    lax.fori_loop(0, _PPW // 16, deint, 0)
    scp.wait()
    pltpu.sync_copy(srows, self_o.at[pl.ds(nbase, _NODES_PW)])

    def issue(c, s):
      g = c * _CP
      pltpu.async_copy(r2e_h.at[r1a.at[pl.ds(g, _CP)]], b1v.at[s],
                       gsem[s][0])
      pltpu.async_copy(r2e_h.at[r2a.at[pl.ds(g, _CP)]], b2v.at[s],
                       gsem[s][1])
      pltpu.async_copy(u2e_h.at[nga.at[pl.ds(g, _CP)]], b3v.at[s],
                       gsem[s][2])

    def wait_gathers(s):
      pltpu.make_async_copy(r2e_h.at[r1a.at[pl.ds(0, _CP)]], b1v.at[s],
                            gsem[s][0]).wait()
      pltpu.make_async_copy(r2e_h.at[r2a.at[pl.ds(0, _CP)]], b2v.at[s],
                            gsem[s][1]).wait()
      pltpu.make_async_copy(u2e_h.at[nga.at[pl.ds(0, _CP)]], b3v.at[s],
                            gsem[s][2]).wait()

    def writeout(c, s):
      g = pbase + c * _CP
      pltpu.async_copy(b1v.at[s], r1_o.at[pl.ds(g, _CP)], osem[s][0])
      pltpu.async_copy(b2v.at[s], r2_o.at[pl.ds(g, _CP)], osem[s][1])
      pltpu.async_copy(b3v.at[s], ng_o.at[pl.ds(g, _CP)], osem[s][2])

    def wait_out(s):
      pltpu.make_async_copy(b1v.at[s], r1_o.at[pl.ds(0, _CP)],
                            osem[s][0]).wait()
      pltpu.make_async_copy(b2v.at[s], r2_o.at[pl.ds(0, _CP)],
                            osem[s][1]).wait()
      pltpu.make_async_copy(b3v.at[s], ng_o.at[pl.ds(0, _CP)],
                            osem[s][2]).wait()

    def chunk_step(c, s):
      wait_gathers(s)
      writeout(c, s)
      s2 = (s + 2) % 3

      def launch_next():
        pl.when(c + 2 >= 3)(lambda: wait_out(s2))
        issue(c + 2, s2)

      pl.when(c + 2 < _NCHUNK_P)(launch_next)

    issue(0, 0)
    issue(1, 1)

    def body(i, _):
      for s in range(3):
        chunk_step(3 * i + s, s)
      return 0

    lax.fori_loop(0, _NCHUNK_P // 3, body, 0)
    for c in range(_NCHUNK_P - _NCHUNK_P % 3, _NCHUNK_P):
      chunk_step(jnp.int32(c), c % 3)
    for s in range(3):
      wait_out(s)

  return k(paths_nat, nodes, u2e, r2e)


def _sc_attrs(attrs_nat, ua2e):
  mesh = plsc.VectorSubcoreMesh(core_axis_name="c", subcore_axis_name="s")

  @functools.partial(
      pl.kernel,
      out_type=jax.ShapeDtypeStruct((BP, D), jnp.float32),
      mesh=mesh,
      compiler_params=pltpu.CompilerParams(use_tc_tiling_on_sc=False,
                                           needs_layout_passes=False),
      scratch_types=[
          pltpu.VMEM((P * A, _NODES_PW), jnp.int32),
          pltpu.VMEM((5, _C * A), jnp.int32),
          pltpu.VMEM((5, _C * A, D), jnp.bfloat16),
          pltpu.VMEM((5, _C, D), jnp.float32),
      ] + [pltpu.SemaphoreType.DMA] * 10,
  )
  def k(attr_h, ua2e_h, at_o, ablk, iav, bav, accv, *sems):
    gsem = sems[0:5]
    osem = sems[5:10]
    wid = lax.axis_index("s") * _NC + lax.axis_index("c")
    pbase = wid * _PPW
    nbase = wid * _NODES_PW

    pltpu.sync_copy(attr_h.at[:, pl.ds(nbase, _NODES_PW)], ablk)

    def issue(c, s):
      g = c * _C
      def build_attr_idx(h, _):
        qq = lax.iota(jnp.int32, 16) + g * A + h * 16
        iav[s, pl.ds(h * 16, 16)] = plsc.load_gather(
            ablk, [qq & (P * A - 1), qq >> 9])
        return 0

      lax.fori_loop(0, _C * A // 16, build_attr_idx, 0)
      pltpu.async_copy(ua2e_h.at[iav.at[s]], bav.at[s], gsem[s])

    def wait_gathers(s):
      pltpu.make_async_copy(ua2e_h.at[iav.at[s]], bav.at[s], gsem[s]).wait()

    def process(s):
      def path_body(p, _):
        base = p * A
        rowi = lax.iota(jnp.int32, 16) * 0 + p
        for c2 in range(D // 32):
          col = pl.ds(c2 * 32, 32)
          acc_a, acc_b = plsc.unpack(
              bav[s, base, col], format=plsc.PackFormat.INTERLEAVED,
              preferred_element_type=jnp.float32)
          for r in range(1, A):
            va, vb = plsc.unpack(
                bav[s, base + r, col], format=plsc.PackFormat.INTERLEAVED,
                preferred_element_type=jnp.float32)
            acc_a = acc_a + va
            acc_b = acc_b + vb
          ii = lax.iota(jnp.int32, 16) * 2 + c2 * 32
          plsc.store_scatter(accv.at[s], [rowi, ii], acc_a)
          plsc.store_scatter(accv.at[s], [rowi, ii + 1], acc_b)
        return 0

      lax.fori_loop(0, _C, path_body, 0)

    def writeout(c, s):
      g = pbase + c * _C
      pltpu.async_copy(accv.at[s], at_o.at[pl.ds(g, _C)], osem[s])

    def wait_out(s):
      pltpu.make_async_copy(accv.at[s], at_o.at[pl.ds(0, _C)],
                            osem[s]).wait()

    def chunk_step(c, s):
      wait_gathers(s)
      process(s)
      writeout(c, s)
      s2 = (s + 4) % 5

      def launch_next():
        pl.when(c + 4 >= 5)(lambda: wait_out(s2))
        issue(c + 4, s2)

      pl.when(c + 4 < _NCHUNK)(launch_next)

    for i in range(4):
      issue(i, i)

    def body(i, _):
      for s in range(5):
        chunk_step(5 * i + s, s)
      return 0

    lax.fori_loop(0, _NCHUNK // 5, body, 0)
    for c in range(_NCHUNK - _NCHUNK % 5, _NCHUNK):
      chunk_step(jnp.int32(c), c % 5)
    for s in range(5):
      wait_out(s)

  return k(attrs_nat, ua2e)


_NB = 128
_R = _NB * P


_R2 = _NB * P // 2


def _tc_body(r1_ref, r2_ref, ng_ref, at_ref, self_ref, w1_ref, b1_ref,
             w2_ref, b2_ref, wa1_ref, ba1_ref, wa2_ref, ba2_ref, wa3_ref,
             out_ref):
  f32 = jnp.float32

  def dot(a, b):
    return jnp.dot(a.astype(jnp.bfloat16), b.astype(jnp.bfloat16),
                   preferred_element_type=f32)

  rr = 2 * _R2

  def unpair(ref):
    x = ref[...]
    return jnp.concatenate([x[:, 0:D], x[:, D:2 * D]], axis=0)

  x1, x2, x3, x4 = (unpair(r1_ref), unpair(r2_ref), unpair(ng_ref),
                    unpair(at_ref))
  w1 = w1_ref[...]
  h1 = (dot(x1, w1[0:D, :]) + dot(x2, w1[D:2 * D, :]) +
        dot(x3, w1[2 * D:3 * D, :]) + dot(x4, w1[3 * D:4 * D, :]) +
        b1_ref[...])
  h1 = jnp.maximum(h1, 0.0)
  o = jnp.maximum(dot(h1, w2_ref[...]) + b2_ref[...], 0.0)

  node_of = lambda r: (r % _R2) // (P // 2)
  ind = (node_of(lax.broadcasted_iota(jnp.int32, (_NB, rr), 1)) ==
         lax.broadcasted_iota(jnp.int32, (_NB, rr), 0)).astype(f32)
  indT = (node_of(lax.broadcasted_iota(jnp.int32, (rr, _NB), 0)) ==
          lax.broadcasted_iota(jnp.int32, (rr, _NB), 1)).astype(f32)

  wa1 = wa1_ref[...]
  self_w = dot(self_ref[...], wa1[D:2 * D, :])
  a1 = jnp.maximum(dot(o, wa1[0:D, :]) + dot(indT, self_w) + ba1_ref[...],
                   0.0)
  a2 = jnp.maximum(dot(a1, wa2_ref[...]) + ba2_ref[...], 0.0)
  logit = dot(a2, wa3_ref[...])
  e = jnp.exp(logit - jnp.max(logit))
  num = dot(ind, o * e)
  den = dot(ind, e)
  out_ref[...] = num / den


def _tc_dense(r1_es, r2_es, ng_es, at_es, self_e, W1, b1, W2, b2, Wa1, ba1,
              Wa2, ba2, Wa3):
  grid = (B // _NB,)
  pair_spec = pl.BlockSpec((_R2, 2 * D), lambda i: (i, 0))
  node_spec = pl.BlockSpec((_NB, D), lambda i: (i, 0))

  def full(shape):
    return pl.BlockSpec(shape, lambda i: tuple(0 for _ in shape))

  return pl.pallas_call(
      _tc_body,
      grid=grid,
      in_specs=[
          pair_spec, pair_spec, pair_spec, pair_spec, node_spec,
          full((4 * D, 2 * D)), full((1, 2 * D)),
          full((2 * D, D)), full((1, D)),
          full((2 * D, D)), full((1, D)),
          full((D, D)), full((1, D)),
          full((D, 1)),
      ],
      out_specs=node_spec,
      out_shape=jax.ShapeDtypeStruct((B, D), jnp.float32),
  )(r1_es.reshape(BP // 2, 2 * D), r2_es.reshape(BP // 2, 2 * D),
    ng_es.reshape(BP // 2, 2 * D), at_es.reshape(BP // 2, 2 * D),
    self_e, W1, b1.reshape(1, -1), W2, b2.reshape(1, -1), Wa1,
    ba1.reshape(1, -1), Wa2, ba2.reshape(1, -1), Wa3)


@jax.jit
def kernel(nodes, nodes_l2paths, nodes_l2n_attrs, u2e, r2e, ua2e, W1, b1,
           W2, b2, Wa1, ba1, Wa2, ba2, Wa3, ba3):
  paths_nat = nodes_l2paths.transpose(2, 1, 0).reshape(3 * P, B).astype(
      jnp.int32)
  attrs_nat = nodes_l2n_attrs.transpose(1, 2, 0).reshape(P * A, B).astype(
      jnp.int32)
  nodes32 = nodes.reshape(-1).astype(jnp.int32)
  u2e_l = u2e.reshape(-1).reshape(N_U, D)
  r2e_l = r2e.reshape(-1).reshape(N_R, D)
  ua2e_l = ua2e.astype(jnp.bfloat16).reshape(-1).reshape(N_A, D)

  r1_es, r2_es, ng_es, self_e = _sc_paths(paths_nat, nodes32, u2e_l, r2e_l)
  attrs_nat = lax.optimization_barrier((attrs_nat, r1_es[0, 0]))[0]
  at_es = _sc_attrs(attrs_nat, ua2e_l)
  del ba3
  return _tc_dense(r1_es, r2_es, ng_es, at_es, self_e, W1, b1, W2, b2,
                   Wa1, ba1, Wa2, ba2, Wa3)
